# Initial kernel scaffold; baseline (speedup 1.0000x reference)
import functools

import jax
import jax.numpy as jnp
from jax import lax
from jax.experimental import pallas as pl
from jax.experimental.pallas import tpu as pltpu
from jax.experimental.pallas import tpu_sc as plsc

NUM_GLYPHS = 5976
LUT_PAD = 6016
ENT_PAD = 2048
NGRP = 13
D = 64
NC, NS = 2, 16
NW = NC * NS
CH = 128


def _prep_body(ent_ref, grp_ref, out_ref):
    out_ref[...] = ent_ref[...] + grp_ref[0]


_prep = pl.pallas_call(
    _prep_body,
    grid=(NGRP,),
    in_specs=[
        pl.BlockSpec((ENT_PAD, D), lambda j: (0, 0)),
        pl.BlockSpec((1, 1, D), lambda j: (j, 0, 0)),
    ],
    out_specs=pl.BlockSpec((ENT_PAD, D), lambda j: (j, 0)),
    out_shape=jax.ShapeDtypeStruct((NGRP * ENT_PAD, D), jnp.float32),
)


def _make_lookup(n_total):
    npw = n_total // NW
    nch = npw // CH
    mesh = plsc.VectorSubcoreMesh(
        core_axis_name="c", subcore_axis_name="s",
        num_cores=NC, num_subcores=NS)

    @functools.partial(
        pl.kernel, mesh=mesh,
        out_type=jax.ShapeDtypeStruct((n_total, D), jnp.float32),
        scratch_types=[
            pltpu.VMEM((npw,), jnp.int32),
            pltpu.VMEM((LUT_PAD,), jnp.int32),
            pltpu.VMEM((LUT_PAD,), jnp.int32),
            pltpu.VMEM((CH,), jnp.int32),
            pltpu.VMEM((CH, D), jnp.float32),
            pltpu.SemaphoreType.DMA,
        ],
    )
    def lookup(ct_hbm, elut_hbm, glut_hbm, gl_hbm, out_hbm,
               gl_v, elut_v, glut_v, idx_v, rows_v, sem):
        wid = lax.axis_index("s") * NC + lax.axis_index("c")
        base = pl.multiple_of(wid * npw, npw)
        pltpu.sync_copy(gl_hbm.at[pl.ds(base, npw)], gl_v)
        pltpu.sync_copy(elut_hbm, elut_v)
        pltpu.sync_copy(glut_hbm, glut_v)

        def chunk(j, carry):
            off = pl.multiple_of(j * CH, CH)
            for t in range(CH // 16):
                g = gl_v[pl.ds(off + t * 16, 16)]
                ge = plsc.load_gather(elut_v, [g])
                gg = plsc.load_gather(glut_v, [g])
                idx_v[pl.ds(t * 16, 16)] = gg * ENT_PAD + ge
            pltpu.async_copy(ct_hbm.at[idx_v], rows_v, sem).wait()
            pltpu.sync_copy(rows_v, out_hbm.at[pl.ds(base + off, CH)])
            return carry

        lax.fori_loop(0, nch, chunk, 0)

    return lookup


def kernel(glyphs, entity_lut, group_lut, entity_table, group_table):
    b, s = glyphs.shape
    n_total = b * s
    gl = glyphs.astype(jnp.int32).reshape(n_total)
    elut = jnp.pad(entity_lut.astype(jnp.int32), (0, LUT_PAD - NUM_GLYPHS))
    glut = jnp.pad(group_lut.astype(jnp.int32), (0, LUT_PAD - NUM_GLYPHS))
    ent_p = jnp.pad(entity_table,
                    ((0, ENT_PAD - entity_table.shape[0]), (0, 0)))
    grp3 = group_table.reshape(NGRP, 1, D)
    ctable = _prep(ent_p, grp3)
    out = _make_lookup(n_total)(ctable, elut, glut, gl)
    return out.reshape(b, s, D)


# speedup vs baseline: 18.1065x; 18.1065x over previous
import functools

import jax
import jax.numpy as jnp
from jax import lax
from jax.experimental import pallas as pl
from jax.experimental.pallas import tpu as pltpu
from jax.experimental.pallas import tpu_sc as plsc

NUM_GLYPHS = 5976
LUT_PAD = 6016
ENT_PAD = 2048
NGRP = 13
D = 64
NC, NS = 2, 16
NW = NC * NS
CH = 128


def _prep_body(ent_ref, grp_ref, out_ref):
    out_ref[...] = ent_ref[...] + grp_ref[0]


_prep = pl.pallas_call(
    _prep_body,
    grid=(NGRP,),
    in_specs=[
        pl.BlockSpec((ENT_PAD, D), lambda j: (0, 0)),
        pl.BlockSpec((1, 1, D), lambda j: (j, 0, 0)),
    ],
    out_specs=pl.BlockSpec((ENT_PAD, D), lambda j: (j, 0)),
    out_shape=jax.ShapeDtypeStruct((NGRP * ENT_PAD, D), jnp.float32),
)


def _make_lookup(n_total):
    npw = n_total // NW
    nch = npw // CH
    mesh = plsc.VectorSubcoreMesh(
        core_axis_name="c", subcore_axis_name="s",
        num_cores=NC, num_subcores=NS)

    @functools.partial(
        pl.kernel, mesh=mesh,
        compiler_params=pltpu.CompilerParams(
            needs_layout_passes=False, use_tc_tiling_on_sc=False),
        out_type=jax.ShapeDtypeStruct((n_total, D), jnp.float32),
        scratch_types=[
            pltpu.VMEM((npw,), jnp.int32),
            pltpu.VMEM((LUT_PAD,), jnp.int32),
            pltpu.VMEM((LUT_PAD,), jnp.int32),
            pltpu.VMEM((CH,), jnp.int32),
            pltpu.VMEM((CH, D), jnp.float32),
            pltpu.SemaphoreType.DMA,
        ],
    )
    def lookup(ct_hbm, elut_hbm, glut_hbm, gl_hbm, out_hbm,
               gl_v, elut_v, glut_v, idx_v, rows_v, sem):
        wid = lax.axis_index("s") * NC + lax.axis_index("c")
        base = pl.multiple_of(wid * npw, npw)
        pltpu.sync_copy(gl_hbm.at[pl.ds(base, npw)], gl_v)
        pltpu.sync_copy(elut_hbm, elut_v)
        pltpu.sync_copy(glut_hbm, glut_v)

        def chunk(j, carry):
            off = pl.multiple_of(j * CH, CH)
            for t in range(CH // 16):
                g = gl_v[pl.ds(off + t * 16, 16)]
                ge = plsc.load_gather(elut_v, [g])
                gg = plsc.load_gather(glut_v, [g])
                idx_v[pl.ds(t * 16, 16)] = gg * ENT_PAD + ge
            pltpu.async_copy(ct_hbm.at[idx_v], rows_v, sem).wait()
            pltpu.sync_copy(rows_v, out_hbm.at[pl.ds(base + off, CH)])
            return carry

        lax.fori_loop(0, nch, chunk, 0)

    return lookup


def kernel(glyphs, entity_lut, group_lut, entity_table, group_table):
    b, s = glyphs.shape
    n_total = b * s
    gl = glyphs.astype(jnp.int32).reshape(n_total)
    elut = jnp.pad(entity_lut.astype(jnp.int32), (0, LUT_PAD - NUM_GLYPHS))
    glut = jnp.pad(group_lut.astype(jnp.int32), (0, LUT_PAD - NUM_GLYPHS))
    ent_p = jnp.pad(entity_table,
                    ((0, ENT_PAD - entity_table.shape[0]), (0, 0)))
    grp3 = group_table.reshape(NGRP, 1, D)
    ctable = _prep(ent_p, grp3)
    out = _make_lookup(n_total)(ctable, elut, glut, gl)
    return out.reshape(b, s, D)
